# baseline (speedup 1.0000x reference)
import functools

import jax
import jax.numpy as jnp
from jax import lax
from jax.experimental import pallas as pl
from jax.experimental.pallas import tpu as pltpu
from jax.experimental.pallas import tpu_sc as plsc

_NC = 2
_NS = 16
_NW = _NC * _NS
_L = 16


def _lane_total(v):
    lanes = lax.iota(jnp.int32, _L)
    for sh in (8, 4, 2, 1):
        v = v + v.at[lanes ^ sh].get(mode="promise_in_bounds")
    return v


def _ln_rows(xb, pb, gb, bb, ob, nrows, d):
    nv = d // _L

    def row_body(i, carry):
        del carry
        e = []
        for k in range(nv):
            e.append(xb[i, pl.ds(k * _L, _L)] + pb[i, pl.ds(k * _L, _L)])
        s = (e[0] + e[1]) + (e[2] + e[3])
        mean = _lane_total(s) * (1.0 / d)
        c = [ek - mean for ek in e]
        q = (c[0] * c[0] + c[1] * c[1]) + (c[2] * c[2] + c[3] * c[3])
        t = _lane_total(q) * (1.0 / d) + 1e-5
        bits = lax.bitcast_convert_type(t, jnp.int32)
        seed = lax.bitcast_convert_type(
            jnp.int32(0x5F3759DF) - lax.shift_right_logical(bits, 1),
            jnp.float32,
        )
        y = seed
        for _ in range(3):
            y = y * (1.5 - 0.5 * t * y * y)
        for k in range(nv):
            gk = gb[pl.ds(k * _L, _L)]
            bk = bb[pl.ds(k * _L, _L)]
            ob[i, pl.ds(k * _L, _L)] = c[k] * (y * gk) + bk
        return 0

    lax.fori_loop(0, nrows, row_body, 0, unroll=4)


def _sc_body(x_hbm, pe_hbm, g_hbm, b_hbm, o_hbm,
             peb, gb, bb, xb, ob, sem_in, sem_out):
    b, nf, s, d = x_hbm.shape
    nslab = b * nf
    stripe = s // _NW

    wid = lax.axis_index("c") * _NS + lax.axis_index("s")
    r0 = wid * stripe

    pltpu.sync_copy(pe_hbm.at[pl.ds(r0, stripe)], peb)
    pltpu.sync_copy(g_hbm, gb)
    pltpu.sync_copy(b_hbm, bb)

    def in_copy(slab, buf):
        bi = slab // nf
        fi = lax.rem(slab, nf)
        return pltpu.make_async_copy(
            x_hbm.at[bi, fi, pl.ds(r0, stripe)], xb.at[buf], sem_in)

    def out_copy(slab, buf):
        bi = slab // nf
        fi = lax.rem(slab, nf)
        return pltpu.make_async_copy(
            ob.at[buf], o_hbm.at[bi, fi, pl.ds(r0, stripe)], sem_out)

    in_copy(0, 0).start()

    def slab_body(i, carry):
        del carry
        par = lax.rem(i, 2)

        @pl.when(i + 1 < nslab)
        def _():
            in_copy(i + 1, 1 - par).start()

        in_copy(i, par).wait()

        @pl.when(i >= 2)
        def _():
            out_copy(i - 2, par).wait()

        _ln_rows(xb.at[par], peb, gb, bb, ob.at[par], stripe, d)
        out_copy(i, par).start()
        return 0

    lax.fori_loop(0, nslab, slab_body, 0)

    out_copy(nslab - 2, lax.rem(nslab - 2, 2)).wait()
    out_copy(nslab - 1, lax.rem(nslab - 1, 2)).wait()


def kernel(x, pos_embed, gamma, beta, batch_size):
    del batch_size
    b, nf, s, d = x.shape
    stripe = s // _NW

    mesh = plsc.VectorSubcoreMesh(core_axis_name="c", subcore_axis_name="s")
    f = pl.kernel(
        _sc_body,
        out_type=jax.ShapeDtypeStruct((b, nf, s, d), jnp.float32),
        mesh=mesh,
        scratch_types=[
            pltpu.VMEM((stripe, d), jnp.float32),
            pltpu.VMEM((d,), jnp.float32),
            pltpu.VMEM((d,), jnp.float32),
            pltpu.VMEM((2, stripe, d), jnp.float32),
            pltpu.VMEM((2, stripe, d), jnp.float32),
            pltpu.SemaphoreType.DMA,
            pltpu.SemaphoreType.DMA,
        ],
    )
    return f(x, pos_embed, gamma, beta)


# speedup vs baseline: 2.9084x; 2.9084x over previous
import functools

import jax
import jax.numpy as jnp
from jax import lax
from jax.experimental import pallas as pl
from jax.experimental.pallas import tpu as pltpu
from jax.experimental.pallas import tpu_sc as plsc

_NC = 2
_NS = 16
_NW = _NC * _NS
_L = 16


def _lane_total(v):
    lanes = lax.iota(jnp.int32, _L)
    for sh in (8, 4, 2, 1):
        v = v + v.at[lanes ^ sh].get(mode="promise_in_bounds")
    return v


def _ln_rows(xb, pb, gb, bb, ob, nrows, d):
    nv = d // _L

    @functools.partial(plsc.parallel_loop, 0, nrows, unroll=8)
    def row_body(i):
        e = []
        for k in range(nv):
            e.append(xb[i, pl.ds(k * _L, _L)] + pb[i, pl.ds(k * _L, _L)])
        s = (e[0] + e[1]) + (e[2] + e[3])
        mean = _lane_total(s) * (1.0 / d)
        c = [ek - mean for ek in e]
        q = (c[0] * c[0] + c[1] * c[1]) + (c[2] * c[2] + c[3] * c[3])
        t = _lane_total(q) * (1.0 / d) + 1e-5
        bits = lax.bitcast_convert_type(t, jnp.int32)
        seed = lax.bitcast_convert_type(
            jnp.int32(0x5F3759DF) - lax.shift_right_logical(bits, 1),
            jnp.float32,
        )
        y = seed
        for _ in range(3):
            y = y * (1.5 - 0.5 * t * y * y)
        for k in range(nv):
            gk = gb[pl.ds(k * _L, _L)]
            bk = bb[pl.ds(k * _L, _L)]
            ob[i, pl.ds(k * _L, _L)] = c[k] * (y * gk) + bk


def _sc_body(x_hbm, pe_hbm, g_hbm, b_hbm, o_hbm,
             peb, gb, bb, xb, ob, sem_in, sem_out):
    b, nf, s, d = x_hbm.shape
    nslab = b * nf
    stripe = s // _NW

    wid = lax.axis_index("c") * _NS + lax.axis_index("s")
    r0 = wid * stripe

    pltpu.sync_copy(pe_hbm.at[pl.ds(r0, stripe)], peb)
    pltpu.sync_copy(g_hbm, gb)
    pltpu.sync_copy(b_hbm, bb)

    def in_copy(slab, buf):
        bi = slab // nf
        fi = lax.rem(slab, nf)
        return pltpu.make_async_copy(
            x_hbm.at[bi, fi, pl.ds(r0, stripe)], xb.at[buf], sem_in)

    def out_copy(slab, buf):
        bi = slab // nf
        fi = lax.rem(slab, nf)
        return pltpu.make_async_copy(
            ob.at[buf], o_hbm.at[bi, fi, pl.ds(r0, stripe)], sem_out)

    in_copy(0, 0).start()

    def slab_body(i, carry):
        del carry
        par = lax.rem(i, 2)

        @pl.when(i + 1 < nslab)
        def _():
            in_copy(i + 1, 1 - par).start()

        in_copy(i, par).wait()

        @pl.when(i >= 2)
        def _():
            out_copy(i - 2, par).wait()

        _ln_rows(xb.at[par], peb, gb, bb, ob.at[par], stripe, d)
        out_copy(i, par).start()
        return 0

    lax.fori_loop(0, nslab, slab_body, 0)

    out_copy(nslab - 2, lax.rem(nslab - 2, 2)).wait()
    out_copy(nslab - 1, lax.rem(nslab - 1, 2)).wait()


def kernel(x, pos_embed, gamma, beta, batch_size):
    del batch_size
    b, nf, s, d = x.shape
    stripe = s // _NW

    mesh = plsc.VectorSubcoreMesh(core_axis_name="c", subcore_axis_name="s")
    f = pl.kernel(
        _sc_body,
        out_type=jax.ShapeDtypeStruct((b, nf, s, d), jnp.float32),
        mesh=mesh,
        scratch_types=[
            pltpu.VMEM((stripe, d), jnp.float32),
            pltpu.VMEM((d,), jnp.float32),
            pltpu.VMEM((d,), jnp.float32),
            pltpu.VMEM((2, stripe, d), jnp.float32),
            pltpu.VMEM((2, stripe, d), jnp.float32),
            pltpu.SemaphoreType.DMA,
            pltpu.SemaphoreType.DMA,
        ],
    )
    return f(x, pos_embed, gamma, beta)
